# Initial kernel scaffold; baseline (speedup 1.0000x reference)
import jax
import jax.numpy as jnp
from jax.experimental import pallas as pl


def kernel(ir, vis, lan, vis_w, vis_b, vis_g, vis_be, ir_w, ir_b, ir_g, ir_be, t_w, t_b, vout_w, vout_b, tout_w, tout_b, W1, b1, W2, b2, W4, W5, ml_w, ml_b, ml2_w, ml2_b, scalars):
    raise NotImplementedError("write your pallas kernel here")


# speedup vs baseline: 2.4207x; 2.4207x over previous
import functools

import jax
import jax.numpy as jnp
from jax import lax
from jax.experimental import pallas as pl
from jax.experimental.pallas import tpu as pltpu

_C = 1024
_NTOK = 100
_MTOK = 20
_KNN = 8
_BIG = 1.0e6
_EPS = 1e-8
_SQC = 32.0

_params = pltpu.CompilerParams if hasattr(pltpu, "CompilerParams") else pltpu.TPUCompilerParams


def _softmax(x, axis):
    m = jnp.max(x, axis=axis, keepdims=True)
    e = jnp.exp(x - m)
    return e / jnp.sum(e, axis=axis, keepdims=True)


def _build_knn_graph(X, P):
    sq = jnp.sum(X * X, axis=1, keepdims=True)
    sqr = sq.T
    G = lax.dot_general(X, X, (((1,), (1,)), ((), ())),
                        preferred_element_type=jnp.float32)
    D = jnp.sqrt(jnp.maximum(sq + sqr - 2.0 * G, 0.0))
    ic = lax.broadcasted_iota(jnp.int32, (P, P), 1)
    ir = lax.broadcasted_iota(jnp.int32, (P, P), 0)
    diag = ir == ic
    Dm = jnp.where(diag, D + _BIG, D)
    Dw = Dm
    A = jnp.full((P, P), _BIG, jnp.float32)
    for _ in range(_KNN):
        m = jnp.min(Dw, axis=1, keepdims=True)
        amin = jnp.min(jnp.where(Dw == m, ic, P), axis=1, keepdims=True)
        sel = ic == amin
        A = jnp.where(sel, Dm, A)
        Dw = jnp.where(sel, _BIG, Dw)
    Wg = jnp.minimum(A, A.T)
    return jnp.where(diag, 0.0, Wg)


def _fw_loop(ws_ref, G, P):
    def body(k, carry):
        for g in range(G):
            row = ws_ref[g, pl.ds(k, 1), :]
            col = row.T
            ws_ref[g] = jnp.minimum(ws_ref[g], col + row)
        return carry
    lax.fori_loop(0, P, body, 0)


def _minmax(t):
    mn = jnp.min(t, axis=1, keepdims=True).min(axis=0, keepdims=True)
    mx = jnp.max(t, axis=1, keepdims=True).max(axis=0, keepdims=True)
    return (t - mn) / (mx - mn + _EPS)


def _sig(x):
    return jax.nn.sigmoid(x)


def _sv(s):
    return jnp.full((1, 1), s, jnp.float32)


def _alpha_beta(md, sigma_v, W1, b1, scalars_ref):
    lam1s = _sig(_sv(scalars_ref[0]))
    mfix = jnp.exp(-(md * md) / (2.0 * sigma_v * sigma_v))
    mlearn = lam1s * mfix + (1.0 - lam1s) * (W1 * mfix + b1)
    mgap = jnp.sum(jnp.sum(mlearn, axis=1, keepdims=True), axis=0,
                   keepdims=True) / 10000.0
    alpha = _sig(_sv(scalars_ref[8]) * mgap + _sv(scalars_ref[9]))
    beta = _sig(_sv(scalars_ref[10]) * mgap + _sv(scalars_ref[11]))
    return alpha, beta


def _proj_img_body(x_ref, w_ref, b_ref, g_ref, be_ref, o_ref):
    bt = x_ref.shape[1]
    for t in range(bt):
        x = x_ref[0, t]
        y = lax.dot_general(x, w_ref[0], (((1,), (1,)), ((), ())),
                            preferred_element_type=jnp.float32)
        y = y + b_ref[0]
        mu = jnp.mean(y, axis=0, keepdims=True)
        var = jnp.mean((y - mu) ** 2, axis=0, keepdims=True)
        y = (y - mu) / jnp.sqrt(var + 1e-5) * g_ref[0] + be_ref[0]
        o_ref[0, t] = jnp.maximum(y, 0.0)


def _proj_txt_body(x_ref, w_ref, b_ref, o_ref):
    bt = x_ref.shape[1]
    for t in range(bt):
        x = x_ref[0, t]
        y = lax.dot_general(x, w_ref[0], (((1,), (1,)), ((), ())),
                            preferred_element_type=jnp.float32)
        o_ref[0, t] = jnp.maximum(y + b_ref[0], 0.0)


def _geo_vi_body(vis_ref, ir_ref, o_ref, ws_ref):
    G = vis_ref.shape[0]
    for g in range(G):
        X = jnp.concatenate([vis_ref[g], ir_ref[g]], axis=0)
        ws_ref[g] = _build_knn_graph(X, 2 * _NTOK)
    _fw_loop(ws_ref, G, 2 * _NTOK)
    for g in range(G):
        o_ref[g] = _minmax(ws_ref[g, 0:_NTOK, _NTOK:2 * _NTOK])


def _median_body(x_ref, o_ref, *, n_elem, r0, r1):
    one = jnp.int32(1)

    def count_ge(mid):
        v = lax.bitcast_convert_type(mid, jnp.float32)
        c = jnp.where(x_ref[...] <= v, 1.0, 0.0)
        return jnp.sum(jnp.sum(c, axis=1, keepdims=True), axis=0, keepdims=True)

    def body(i, carry):
        lo0, hi0, lo1, hi1 = carry
        mid0 = (lo0 + hi0) >> one
        mid1 = (lo1 + hi1) >> one
        p0 = count_ge(mid0) >= float(r0 + 1)
        p1 = count_ge(mid1) >= float(r1 + 1)
        hi0 = jnp.where(p0, mid0, hi0)
        lo0 = jnp.where(p0, lo0, mid0 + one)
        hi1 = jnp.where(p1, mid1, hi1)
        lo1 = jnp.where(p1, lo1, mid1 + one)
        return lo0, hi0, lo1, hi1

    lo = jnp.zeros((1, 1), jnp.int32)
    hi = jnp.full((1, 1), 0x3F800000, jnp.int32)
    lo0, _, lo1, _ = lax.fori_loop(0, 31, body, (lo, hi, lo, hi))
    f0 = lax.bitcast_convert_type(lo0, jnp.float32)
    f1 = lax.bitcast_convert_type(lo1, jnp.float32)
    o_ref[0] = ((f0 + f1) * 0.5)[0, 0]


def _geo_it_body(vis_ref, ir_ref, lan_ref, mvd_ref, W1_ref, b1_ref,
                 sigma_ref, scalars_ref, o_ref, ws_ref):
    G = vis_ref.shape[0]
    sigma_v = _sv(sigma_ref[0])
    for g in range(G):
        alpha, beta = _alpha_beta(mvd_ref[g], sigma_v, W1_ref[...],
                                  b1_ref[...], scalars_ref)
        feat = alpha * vis_ref[g] + beta * ir_ref[g]
        X = jnp.concatenate([feat, lan_ref[g]], axis=0)
        ws_ref[g] = _build_knn_graph(X, _NTOK + _MTOK)
    _fw_loop(ws_ref, G, _NTOK + _MTOK)
    for g in range(G):
        o_ref[g] = _minmax(ws_ref[g, _NTOK:_NTOK + _MTOK, 0:_NTOK])


def _final_body(vis_ref, ir_ref, lan_ref, pimg_ref, ptxt_ref, mvd_ref, mtd_ref,
                W1_ref, b1_ref, W2_ref, b2_ref, W4_ref, W5_ref,
                mlw_ref, mlb_ref, ml2w_ref, ml2b_ref,
                voutw_ref, voutb_ref, toutw_ref, toutb_ref,
                sigma_ref, sigma2_ref, scalars_ref,
                ovis_ref, olan_ref, oparts_ref):
    f32 = jnp.float32
    dotT = lambda a, b: lax.dot_general(a, b, (((1,), (1,)), ((), ())),
                                        preferred_element_type=f32)
    dot = lambda a, b: lax.dot_general(a, b, (((1,), (0,)), ((), ())),
                                       preferred_element_type=f32)
    sc = scalars_ref
    a1, be1, g1 = _sv(sc[2]), _sv(sc[3]), _sv(sc[4])
    a2, be2, g2 = _sv(sc[5]), _sv(sc[6]), _sv(sc[7])

    alpha, beta = _alpha_beta(mvd_ref[0], _sv(sigma_ref[0]), W1_ref[...],
                              b1_ref[...], sc)
    feat = alpha * vis_ref[0] + beta * ir_ref[0]
    Qf = alpha * pimg_ref[0, 0] + beta * pimg_ref[3, 0]
    Kf = alpha * pimg_ref[1, 0] + beta * pimg_ref[4, 0]
    Vf = alpha * pimg_ref[2, 0] + beta * pimg_ref[5, 0]
    Qt, Kt, Vt = ptxt_ref[0, 0], ptxt_ref[1, 0], ptxt_ref[2, 0]
    lan = lan_ref[0]

    s2 = _sv(sigma2_ref[0])
    mtd = mtd_ref[0]
    M_t2i_fix = jnp.exp(-(mtd * mtd) / (2.0 * s2 * s2))
    M_i2t_fix = M_t2i_fix.T
    lam2s = _sig(_sv(sc[1]))
    M_i2t_pre = lam2s * M_i2t_fix + (1.0 - lam2s) * (W2_ref[...] * M_i2t_fix
                                                     + b2_ref[...])
    M_t2i_lo = _softmax(M_i2t_pre.T / _SQC, axis=1)
    M_i2t_lo = _softmax(M_i2t_pre / _SQC, axis=1)
    M_t2i_learn = dotT(dot(M_i2t_lo, lan), mlw_ref[...]) + mlb_ref[...]
    M_i2t_learn = dotT(dot(M_t2i_lo, feat), ml2w_ref[...]) + ml2b_ref[...]

    A = dotT(Qt, Kf) + a1 * dotT(Qt, M_t2i_learn) \
        + be1 * dot(W4_ref[...], dotT(M_t2i_learn, Kf))
    Av = _softmax(A / _SQC, axis=1)
    new_lan = dot(Av, Vf) + g1 * dot(M_t2i_lo, Vf)

    Bm = dotT(Qf, Kt) + a2 * dotT(Qf, M_i2t_learn) \
        + be2 * dot(W5_ref[...], dotT(M_i2t_learn, Kt))
    Bv = _softmax(Bm / _SQC, axis=1)
    new_vis = dot(Bv, Vt) + g2 * dot(M_i2t_lo, Vt)

    ovis_ref[0] = dotT(new_vis, voutw_ref[...]) + voutb_ref[...]
    olan_ref[0] = dotT(new_lan, toutw_ref[...]) + toutb_ref[...]

    fA = _softmax(M_t2i_fix / _SQC, axis=1)
    fB = _softmax(M_i2t_fix / _SQC, axis=1)
    s = lambda x: jnp.sum(jnp.sum(x, axis=1, keepdims=True), axis=0,
                          keepdims=True)
    parts = jnp.concatenate(
        [s(jnp.abs(Av - fA)), s(jnp.abs(Bv - fB)),
         s(Av), s(Av * Av), s(fA), s(fA * fA),
         s(Bv), s(Bv * Bv), s(fB), s(fB * fB),
         jnp.zeros((1, 6), f32)], axis=1)
    oparts_ref[0] = parts


def _vm(nbytes=56 * 1024 * 1024):
    return _params(vmem_limit_bytes=nbytes)


def kernel(ir, vis, lan, vis_w, vis_b, vis_g, vis_be, ir_w, ir_b, ir_g, ir_be,
           t_w, t_b, vout_w, vout_b, tout_w, tout_b, W1, b1, W2, b2, W4, W5,
           ml_w, ml_b, ml2_w, ml2_b, scalars):
    f32 = jnp.float32
    B = ir.shape[0]
    C = _C
    vis_r = vis.reshape(B, C, _NTOK).transpose(0, 2, 1)
    ir_r = ir.reshape(B, C, _NTOK).transpose(0, 2, 1)
    lan_t = lan.transpose(0, 2, 1)

    x2 = jnp.stack([vis_r, ir_r], axis=0)
    w6 = jnp.concatenate([vis_w, ir_w], axis=0)
    b6 = jnp.concatenate([vis_b, ir_b], axis=0).reshape(6, 1, C)
    g6 = jnp.concatenate([vis_g, ir_g], axis=0).reshape(6, 1, C)
    be6 = jnp.concatenate([vis_be, ir_be], axis=0).reshape(6, 1, C)
    bt = 8 if B % 8 == 0 else 1
    pimg = pl.pallas_call(
        _proj_img_body,
        grid=(6, B // bt),
        in_specs=[
            pl.BlockSpec((1, bt, _NTOK, C), lambda j, b: (j // 3, b, 0, 0)),
            pl.BlockSpec((1, C, C), lambda j, b: (j, 0, 0)),
            pl.BlockSpec((1, 1, C), lambda j, b: (j, 0, 0)),
            pl.BlockSpec((1, 1, C), lambda j, b: (j, 0, 0)),
            pl.BlockSpec((1, 1, C), lambda j, b: (j, 0, 0)),
        ],
        out_specs=pl.BlockSpec((1, bt, _NTOK, C), lambda j, b: (j, b, 0, 0)),
        out_shape=jax.ShapeDtypeStruct((6, B, _NTOK, C), f32),
        compiler_params=_vm(),
        name="proj_img",
    )(x2, w6, b6, g6, be6)

    bt2 = 16 if B % 16 == 0 else 1
    ptxt = pl.pallas_call(
        _proj_txt_body,
        grid=(3, B // bt2),
        in_specs=[
            pl.BlockSpec((1, bt2, _MTOK, C), lambda j, b: (0, b, 0, 0)),
            pl.BlockSpec((1, C, C), lambda j, b: (j, 0, 0)),
            pl.BlockSpec((1, 1, C), lambda j, b: (j, 0, 0)),
        ],
        out_specs=pl.BlockSpec((1, bt2, _MTOK, C), lambda j, b: (j, b, 0, 0)),
        out_shape=jax.ShapeDtypeStruct((3, B, _MTOK, C), f32),
        compiler_params=_vm(),
        name="proj_txt",
    )(lan_t.reshape(1, B, _MTOK, C), t_w, t_b.reshape(3, 1, C))

    G = 8 if B % 8 == 0 else 1
    P1 = 2 * _NTOK
    mvd = pl.pallas_call(
        _geo_vi_body,
        grid=(B // G,),
        in_specs=[
            pl.BlockSpec((G, _NTOK, C), lambda b: (b, 0, 0)),
            pl.BlockSpec((G, _NTOK, C), lambda b: (b, 0, 0)),
        ],
        out_specs=pl.BlockSpec((G, _NTOK, _NTOK), lambda b: (b, 0, 0)),
        out_shape=jax.ShapeDtypeStruct((B, _NTOK, _NTOK), f32),
        scratch_shapes=[pltpu.VMEM((G, P1, P1), f32)],
        compiler_params=_vm(),
        name="geo_vi",
    )(vis_r, ir_r)

    n1 = B * _NTOK * _NTOK
    sigma = pl.pallas_call(
        functools.partial(_median_body, n_elem=n1, r0=(n1 - 1) // 2,
                          r1=n1 // 2),
        in_specs=[pl.BlockSpec(memory_space=pltpu.VMEM)],
        out_specs=pl.BlockSpec(memory_space=pltpu.SMEM),
        out_shape=jax.ShapeDtypeStruct((1,), f32),
        compiler_params=_vm(),
        name="median1",
    )(mvd.reshape(n1 // 1000, 1000))

    P2 = _NTOK + _MTOK
    mtd = pl.pallas_call(
        _geo_it_body,
        grid=(B // G,),
        in_specs=[
            pl.BlockSpec((G, _NTOK, C), lambda b: (b, 0, 0)),
            pl.BlockSpec((G, _NTOK, C), lambda b: (b, 0, 0)),
            pl.BlockSpec((G, _MTOK, C), lambda b: (b, 0, 0)),
            pl.BlockSpec((G, _NTOK, _NTOK), lambda b: (b, 0, 0)),
            pl.BlockSpec((_NTOK, _NTOK), lambda b: (0, 0)),
            pl.BlockSpec((_NTOK, _NTOK), lambda b: (0, 0)),
            pl.BlockSpec(memory_space=pltpu.SMEM),
            pl.BlockSpec(memory_space=pltpu.SMEM),
        ],
        out_specs=pl.BlockSpec((G, _MTOK, _NTOK), lambda b: (b, 0, 0)),
        out_shape=jax.ShapeDtypeStruct((B, _MTOK, _NTOK), f32),
        scratch_shapes=[pltpu.VMEM((G, P2, P2), f32)],
        compiler_params=_vm(),
        name="geo_it",
    )(vis_r, ir_r, lan_t, mvd, W1, b1, sigma, scalars)

    n2 = B * _MTOK * _NTOK
    sigma2 = pl.pallas_call(
        functools.partial(_median_body, n_elem=n2, r0=(n2 - 1) // 2,
                          r1=n2 // 2),
        in_specs=[pl.BlockSpec(memory_space=pltpu.VMEM)],
        out_specs=pl.BlockSpec(memory_space=pltpu.SMEM),
        out_shape=jax.ShapeDtypeStruct((1,), f32),
        compiler_params=_vm(),
        name="median2",
    )(mtd.reshape(n2 // 1000, 1000))

    whole = lambda shp: pl.BlockSpec(shp, lambda b: tuple(0 for _ in shp))
    vis_tok, new_lan, parts = pl.pallas_call(
        _final_body,
        grid=(B,),
        in_specs=[
            pl.BlockSpec((1, _NTOK, C), lambda b: (b, 0, 0)),
            pl.BlockSpec((1, _NTOK, C), lambda b: (b, 0, 0)),
            pl.BlockSpec((1, _MTOK, C), lambda b: (b, 0, 0)),
            pl.BlockSpec((6, 1, _NTOK, C), lambda b: (0, b, 0, 0)),
            pl.BlockSpec((3, 1, _MTOK, C), lambda b: (0, b, 0, 0)),
            pl.BlockSpec((1, _NTOK, _NTOK), lambda b: (b, 0, 0)),
            pl.BlockSpec((1, _MTOK, _NTOK), lambda b: (b, 0, 0)),
            whole((_NTOK, _NTOK)), whole((_NTOK, _NTOK)),
            whole((_NTOK, _MTOK)), whole((_NTOK, _MTOK)),
            whole((_MTOK, _NTOK)), whole((_NTOK, _MTOK)),
            whole((C, C)), whole((1, C)), whole((C, C)), whole((1, C)),
            whole((C, C)), whole((1, C)), whole((C, C)), whole((1, C)),
            pl.BlockSpec(memory_space=pltpu.SMEM),
            pl.BlockSpec(memory_space=pltpu.SMEM),
            pl.BlockSpec(memory_space=pltpu.SMEM),
        ],
        out_specs=[
            pl.BlockSpec((1, _NTOK, C), lambda b: (b, 0, 0)),
            pl.BlockSpec((1, _MTOK, C), lambda b: (b, 0, 0)),
            pl.BlockSpec((1, 1, 16), lambda b: (b, 0, 0)),
        ],
        out_shape=[
            jax.ShapeDtypeStruct((B, _NTOK, C), f32),
            jax.ShapeDtypeStruct((B, _MTOK, C), f32),
            jax.ShapeDtypeStruct((B, 1, 16), f32),
        ],
        compiler_params=_vm(),
        name="final_attn",
    )(vis_r, ir_r, lan_t, pimg, ptxt, mvd, mtd,
      W1, b1, W2, b2, W4, W5,
      ml_w, ml_b.reshape(1, C), ml2_w, ml2_b.reshape(1, C),
      vout_w, vout_b.reshape(1, C), tout_w, tout_b.reshape(1, C),
      sigma, sigma2, scalars)

    new_vis = vis_tok.transpose(0, 2, 1).reshape(B, C, 10, 10)

    p = parts[:, 0, :]
    nA = B * _MTOK * _NTOK
    nB = B * _NTOK * _MTOK
    l1 = jnp.sum(p[:, 0]) / nA + jnp.sum(p[:, 1]) / nB
    var = lambda sx, sx2, n: (jnp.sum(sx2) - jnp.sum(sx) ** 2 / n) / (n - 1)
    l2 = -jnp.abs(var(p[:, 2], p[:, 3], nA) - var(p[:, 4], p[:, 5], nA)) \
         - jnp.abs(var(p[:, 6], p[:, 7], nB) - var(p[:, 8], p[:, 9], nB))
    loss = (l1 + 100.0 * l2) * 1000.0
    return new_vis, new_lan, loss


# speedup vs baseline: 3.0827x; 1.2734x over previous
import functools

import jax
import jax.numpy as jnp
from jax import lax
from jax.experimental import pallas as pl
from jax.experimental.pallas import tpu as pltpu

_C = 1024
_NTOK = 100
_MTOK = 20
_KNN = 8
_BIG = 1.0e6
_EPS = 1e-8
_SQC = 32.0

_params = pltpu.CompilerParams if hasattr(pltpu, "CompilerParams") else pltpu.TPUCompilerParams


def _softmax(x, axis):
    m = jnp.max(x, axis=axis, keepdims=True)
    e = jnp.exp(x - m)
    return e / jnp.sum(e, axis=axis, keepdims=True)


def _build_knn_graph(X, P):
    sq = jnp.sum(X * X, axis=1, keepdims=True)
    sqr = sq.T
    G = lax.dot_general(X, X, (((1,), (1,)), ((), ())),
                        preferred_element_type=jnp.float32)
    D = jnp.sqrt(jnp.maximum(sq + sqr - 2.0 * G, 0.0))
    ic = lax.broadcasted_iota(jnp.int32, (P, P), 1)
    ir = lax.broadcasted_iota(jnp.int32, (P, P), 0)
    diag = ir == ic
    Dm = jnp.where(diag, D + _BIG, D)
    Dw = Dm
    A = jnp.full((P, P), _BIG, jnp.float32)
    for _ in range(_KNN):
        m = jnp.min(Dw, axis=1, keepdims=True)
        amin = jnp.min(jnp.where(Dw == m, ic, P), axis=1, keepdims=True)
        sel = ic == amin
        A = jnp.where(sel, Dm, A)
        Dw = jnp.where(sel, _BIG, Dw)
    Wg = jnp.minimum(A, A.T)
    return jnp.where(diag, 0.0, Wg)


def _fw_loop(ws_ref, rt_ref, G, P):
    KB = 8

    def body(kb, carry):
        base = pl.multiple_of(kb * KB, KB)
        for g in range(G):
            R = ws_ref[g, pl.ds(base, KB), :]
            rt_ref[g] = R.T
            Dblk = rt_ref[g, pl.ds(base, KB), :]
            for j in range(KB):
                cb = Dblk[:, j:j + 1]
                R = jnp.minimum(R, cb + R[j:j + 1, :])
                Dblk = jnp.minimum(Dblk, cb + Dblk[j:j + 1, :])
            RT = R.T
            W = ws_ref[g]
            for j in range(KB):
                W = jnp.minimum(W, RT[:, j:j + 1] + R[j:j + 1, :])
            ws_ref[g] = W
        return carry

    lax.fori_loop(0, P // KB, body, 0)


def _minmax(t):
    mn = jnp.min(t, axis=1, keepdims=True).min(axis=0, keepdims=True)
    mx = jnp.max(t, axis=1, keepdims=True).max(axis=0, keepdims=True)
    return (t - mn) / (mx - mn + _EPS)


def _sig(x):
    return jax.nn.sigmoid(x)


def _sv(s):
    return jnp.full((1, 1), s, jnp.float32)


def _alpha_beta(md, sigma_v, W1, b1, scalars_ref):
    lam1s = _sig(_sv(scalars_ref[0]))
    mfix = jnp.exp(-(md * md) / (2.0 * sigma_v * sigma_v))
    mlearn = lam1s * mfix + (1.0 - lam1s) * (W1 * mfix + b1)
    mgap = jnp.sum(jnp.sum(mlearn, axis=1, keepdims=True), axis=0,
                   keepdims=True) / 10000.0
    alpha = _sig(_sv(scalars_ref[8]) * mgap + _sv(scalars_ref[9]))
    beta = _sig(_sv(scalars_ref[10]) * mgap + _sv(scalars_ref[11]))
    return alpha, beta


def _proj_img_body(x_ref, w_ref, b_ref, g_ref, be_ref, o_ref):
    bt = x_ref.shape[1]
    for t in range(bt):
        x = x_ref[0, t]
        y = lax.dot_general(x, w_ref[0], (((1,), (1,)), ((), ())),
                            preferred_element_type=jnp.float32)
        y = y + b_ref[0]
        mu = jnp.mean(y, axis=0, keepdims=True)
        var = jnp.mean((y - mu) ** 2, axis=0, keepdims=True)
        y = (y - mu) / jnp.sqrt(var + 1e-5) * g_ref[0] + be_ref[0]
        o_ref[0, t] = jnp.maximum(y, 0.0)


def _proj_txt_body(x_ref, w_ref, b_ref, o_ref):
    bt = x_ref.shape[1]
    for t in range(bt):
        x = x_ref[0, t]
        y = lax.dot_general(x, w_ref[0], (((1,), (1,)), ((), ())),
                            preferred_element_type=jnp.float32)
        o_ref[0, t] = jnp.maximum(y + b_ref[0], 0.0)


def _geo_vi_body(vis_ref, ir_ref, o_ref, ws_ref, rt_ref):
    G = vis_ref.shape[0]
    for g in range(G):
        X = jnp.concatenate([vis_ref[g], ir_ref[g]], axis=0)
        ws_ref[g] = _build_knn_graph(X, 2 * _NTOK)
    _fw_loop(ws_ref, rt_ref, G, 2 * _NTOK)
    for g in range(G):
        o_ref[g] = _minmax(ws_ref[g, 0:_NTOK, _NTOK:2 * _NTOK])


def _median_body(x_ref, o_ref, *, n_elem, r0, r1):
    one = jnp.int32(1)

    def count_ge(mid):
        v = lax.bitcast_convert_type(mid, jnp.float32)
        c = jnp.where(x_ref[...] <= v, 1.0, 0.0)
        return jnp.sum(jnp.sum(c, axis=1, keepdims=True), axis=0, keepdims=True)

    def body(i, carry):
        lo0, hi0, lo1, hi1 = carry
        mid0 = (lo0 + hi0) >> one
        mid1 = (lo1 + hi1) >> one
        p0 = count_ge(mid0) >= float(r0 + 1)
        p1 = count_ge(mid1) >= float(r1 + 1)
        hi0 = jnp.where(p0, mid0, hi0)
        lo0 = jnp.where(p0, lo0, mid0 + one)
        hi1 = jnp.where(p1, mid1, hi1)
        lo1 = jnp.where(p1, lo1, mid1 + one)
        return lo0, hi0, lo1, hi1

    lo = jnp.zeros((1, 1), jnp.int32)
    hi = jnp.full((1, 1), 0x3F800000, jnp.int32)
    lo0, _, lo1, _ = lax.fori_loop(0, 31, body, (lo, hi, lo, hi))
    f0 = lax.bitcast_convert_type(lo0, jnp.float32)
    f1 = lax.bitcast_convert_type(lo1, jnp.float32)
    o_ref[0] = ((f0 + f1) * 0.5)[0, 0]


def _geo_it_body(vis_ref, ir_ref, lan_ref, mvd_ref, W1_ref, b1_ref,
                 sigma_ref, scalars_ref, o_ref, ws_ref, rt_ref):
    G = vis_ref.shape[0]
    sigma_v = _sv(sigma_ref[0])
    for g in range(G):
        alpha, beta = _alpha_beta(mvd_ref[g], sigma_v, W1_ref[...],
                                  b1_ref[...], scalars_ref)
        feat = alpha * vis_ref[g] + beta * ir_ref[g]
        X = jnp.concatenate([feat, lan_ref[g]], axis=0)
        ws_ref[g] = _build_knn_graph(X, _NTOK + _MTOK)
    _fw_loop(ws_ref, rt_ref, G, _NTOK + _MTOK)
    for g in range(G):
        o_ref[g] = _minmax(ws_ref[g, _NTOK:_NTOK + _MTOK, 0:_NTOK])


def _final_body(vis_ref, ir_ref, lan_ref, pimg_ref, ptxt_ref, mvd_ref, mtd_ref,
                W1_ref, b1_ref, W2_ref, b2_ref, W4_ref, W5_ref,
                mlw_ref, mlb_ref, ml2w_ref, ml2b_ref,
                voutw_ref, voutb_ref, toutw_ref, toutb_ref,
                sigma_ref, sigma2_ref, scalars_ref,
                ovis_ref, olan_ref, oparts_ref):
    f32 = jnp.float32
    dotT = lambda a, b: lax.dot_general(a, b, (((1,), (1,)), ((), ())),
                                        preferred_element_type=f32)
    dot = lambda a, b: lax.dot_general(a, b, (((1,), (0,)), ((), ())),
                                       preferred_element_type=f32)
    sc = scalars_ref
    a1, be1, g1 = _sv(sc[2]), _sv(sc[3]), _sv(sc[4])
    a2, be2, g2 = _sv(sc[5]), _sv(sc[6]), _sv(sc[7])

    alpha, beta = _alpha_beta(mvd_ref[0], _sv(sigma_ref[0]), W1_ref[...],
                              b1_ref[...], sc)
    feat = alpha * vis_ref[0] + beta * ir_ref[0]
    Qf = alpha * pimg_ref[0, 0] + beta * pimg_ref[3, 0]
    Kf = alpha * pimg_ref[1, 0] + beta * pimg_ref[4, 0]
    Vf = alpha * pimg_ref[2, 0] + beta * pimg_ref[5, 0]
    Qt, Kt, Vt = ptxt_ref[0, 0], ptxt_ref[1, 0], ptxt_ref[2, 0]
    lan = lan_ref[0]

    s2 = _sv(sigma2_ref[0])
    mtd = mtd_ref[0]
    M_t2i_fix = jnp.exp(-(mtd * mtd) / (2.0 * s2 * s2))
    M_i2t_fix = M_t2i_fix.T
    lam2s = _sig(_sv(sc[1]))
    M_i2t_pre = lam2s * M_i2t_fix + (1.0 - lam2s) * (W2_ref[...] * M_i2t_fix
                                                     + b2_ref[...])
    M_t2i_lo = _softmax(M_i2t_pre.T / _SQC, axis=1)
    M_i2t_lo = _softmax(M_i2t_pre / _SQC, axis=1)
    M_t2i_learn = dotT(dot(M_i2t_lo, lan), mlw_ref[...]) + mlb_ref[...]
    M_i2t_learn = dotT(dot(M_t2i_lo, feat), ml2w_ref[...]) + ml2b_ref[...]

    A = dotT(Qt, Kf) + a1 * dotT(Qt, M_t2i_learn) \
        + be1 * dot(W4_ref[...], dotT(M_t2i_learn, Kf))
    Av = _softmax(A / _SQC, axis=1)
    new_lan = dot(Av, Vf) + g1 * dot(M_t2i_lo, Vf)

    Bm = dotT(Qf, Kt) + a2 * dotT(Qf, M_i2t_learn) \
        + be2 * dot(W5_ref[...], dotT(M_i2t_learn, Kt))
    Bv = _softmax(Bm / _SQC, axis=1)
    new_vis = dot(Bv, Vt) + g2 * dot(M_i2t_lo, Vt)

    ovis_ref[0] = dotT(new_vis, voutw_ref[...]) + voutb_ref[...]
    olan_ref[0] = dotT(new_lan, toutw_ref[...]) + toutb_ref[...]

    fA = _softmax(M_t2i_fix / _SQC, axis=1)
    fB = _softmax(M_i2t_fix / _SQC, axis=1)
    s = lambda x: jnp.sum(jnp.sum(x, axis=1, keepdims=True), axis=0,
                          keepdims=True)
    parts = jnp.concatenate(
        [s(jnp.abs(Av - fA)), s(jnp.abs(Bv - fB)),
         s(Av), s(Av * Av), s(fA), s(fA * fA),
         s(Bv), s(Bv * Bv), s(fB), s(fB * fB),
         jnp.zeros((1, 6), f32)], axis=1)
    oparts_ref[0] = parts


def _vm(nbytes=56 * 1024 * 1024):
    return _params(vmem_limit_bytes=nbytes)


def kernel(ir, vis, lan, vis_w, vis_b, vis_g, vis_be, ir_w, ir_b, ir_g, ir_be,
           t_w, t_b, vout_w, vout_b, tout_w, tout_b, W1, b1, W2, b2, W4, W5,
           ml_w, ml_b, ml2_w, ml2_b, scalars):
    f32 = jnp.float32
    B = ir.shape[0]
    C = _C
    vis_r = vis.reshape(B, C, _NTOK).transpose(0, 2, 1)
    ir_r = ir.reshape(B, C, _NTOK).transpose(0, 2, 1)
    lan_t = lan.transpose(0, 2, 1)

    x2 = jnp.stack([vis_r, ir_r], axis=0)
    w6 = jnp.concatenate([vis_w, ir_w], axis=0)
    b6 = jnp.concatenate([vis_b, ir_b], axis=0).reshape(6, 1, C)
    g6 = jnp.concatenate([vis_g, ir_g], axis=0).reshape(6, 1, C)
    be6 = jnp.concatenate([vis_be, ir_be], axis=0).reshape(6, 1, C)
    bt = 8 if B % 8 == 0 else 1
    pimg = pl.pallas_call(
        _proj_img_body,
        grid=(6, B // bt),
        in_specs=[
            pl.BlockSpec((1, bt, _NTOK, C), lambda j, b: (j // 3, b, 0, 0)),
            pl.BlockSpec((1, C, C), lambda j, b: (j, 0, 0)),
            pl.BlockSpec((1, 1, C), lambda j, b: (j, 0, 0)),
            pl.BlockSpec((1, 1, C), lambda j, b: (j, 0, 0)),
            pl.BlockSpec((1, 1, C), lambda j, b: (j, 0, 0)),
        ],
        out_specs=pl.BlockSpec((1, bt, _NTOK, C), lambda j, b: (j, b, 0, 0)),
        out_shape=jax.ShapeDtypeStruct((6, B, _NTOK, C), f32),
        compiler_params=_vm(),
        name="proj_img",
    )(x2, w6, b6, g6, be6)

    bt2 = 16 if B % 16 == 0 else 1
    ptxt = pl.pallas_call(
        _proj_txt_body,
        grid=(3, B // bt2),
        in_specs=[
            pl.BlockSpec((1, bt2, _MTOK, C), lambda j, b: (0, b, 0, 0)),
            pl.BlockSpec((1, C, C), lambda j, b: (j, 0, 0)),
            pl.BlockSpec((1, 1, C), lambda j, b: (j, 0, 0)),
        ],
        out_specs=pl.BlockSpec((1, bt2, _MTOK, C), lambda j, b: (j, b, 0, 0)),
        out_shape=jax.ShapeDtypeStruct((3, B, _MTOK, C), f32),
        compiler_params=_vm(),
        name="proj_txt",
    )(lan_t.reshape(1, B, _MTOK, C), t_w, t_b.reshape(3, 1, C))

    G = 8 if B % 8 == 0 else 1
    P1 = 2 * _NTOK
    mvd = pl.pallas_call(
        _geo_vi_body,
        grid=(B // G,),
        in_specs=[
            pl.BlockSpec((G, _NTOK, C), lambda b: (b, 0, 0)),
            pl.BlockSpec((G, _NTOK, C), lambda b: (b, 0, 0)),
        ],
        out_specs=pl.BlockSpec((G, _NTOK, _NTOK), lambda b: (b, 0, 0)),
        out_shape=jax.ShapeDtypeStruct((B, _NTOK, _NTOK), f32),
        scratch_shapes=[pltpu.VMEM((G, P1, P1), f32),
                        pltpu.VMEM((G, P1, 8), f32)],
        compiler_params=_vm(),
        name="geo_vi",
    )(vis_r, ir_r)

    n1 = B * _NTOK * _NTOK
    sigma = pl.pallas_call(
        functools.partial(_median_body, n_elem=n1, r0=(n1 - 1) // 2,
                          r1=n1 // 2),
        in_specs=[pl.BlockSpec(memory_space=pltpu.VMEM)],
        out_specs=pl.BlockSpec(memory_space=pltpu.SMEM),
        out_shape=jax.ShapeDtypeStruct((1,), f32),
        compiler_params=_vm(),
        name="median1",
    )(mvd.reshape(n1 // 1000, 1000))

    P2 = _NTOK + _MTOK
    mtd = pl.pallas_call(
        _geo_it_body,
        grid=(B // G,),
        in_specs=[
            pl.BlockSpec((G, _NTOK, C), lambda b: (b, 0, 0)),
            pl.BlockSpec((G, _NTOK, C), lambda b: (b, 0, 0)),
            pl.BlockSpec((G, _MTOK, C), lambda b: (b, 0, 0)),
            pl.BlockSpec((G, _NTOK, _NTOK), lambda b: (b, 0, 0)),
            pl.BlockSpec((_NTOK, _NTOK), lambda b: (0, 0)),
            pl.BlockSpec((_NTOK, _NTOK), lambda b: (0, 0)),
            pl.BlockSpec(memory_space=pltpu.SMEM),
            pl.BlockSpec(memory_space=pltpu.SMEM),
        ],
        out_specs=pl.BlockSpec((G, _MTOK, _NTOK), lambda b: (b, 0, 0)),
        out_shape=jax.ShapeDtypeStruct((B, _MTOK, _NTOK), f32),
        scratch_shapes=[pltpu.VMEM((G, P2, P2), f32),
                        pltpu.VMEM((G, P2, 8), f32)],
        compiler_params=_vm(),
        name="geo_it",
    )(vis_r, ir_r, lan_t, mvd, W1, b1, sigma, scalars)

    n2 = B * _MTOK * _NTOK
    sigma2 = pl.pallas_call(
        functools.partial(_median_body, n_elem=n2, r0=(n2 - 1) // 2,
                          r1=n2 // 2),
        in_specs=[pl.BlockSpec(memory_space=pltpu.VMEM)],
        out_specs=pl.BlockSpec(memory_space=pltpu.SMEM),
        out_shape=jax.ShapeDtypeStruct((1,), f32),
        compiler_params=_vm(),
        name="median2",
    )(mtd.reshape(n2 // 1000, 1000))

    whole = lambda shp: pl.BlockSpec(shp, lambda b: tuple(0 for _ in shp))
    vis_tok, new_lan, parts = pl.pallas_call(
        _final_body,
        grid=(B,),
        in_specs=[
            pl.BlockSpec((1, _NTOK, C), lambda b: (b, 0, 0)),
            pl.BlockSpec((1, _NTOK, C), lambda b: (b, 0, 0)),
            pl.BlockSpec((1, _MTOK, C), lambda b: (b, 0, 0)),
            pl.BlockSpec((6, 1, _NTOK, C), lambda b: (0, b, 0, 0)),
            pl.BlockSpec((3, 1, _MTOK, C), lambda b: (0, b, 0, 0)),
            pl.BlockSpec((1, _NTOK, _NTOK), lambda b: (b, 0, 0)),
            pl.BlockSpec((1, _MTOK, _NTOK), lambda b: (b, 0, 0)),
            whole((_NTOK, _NTOK)), whole((_NTOK, _NTOK)),
            whole((_NTOK, _MTOK)), whole((_NTOK, _MTOK)),
            whole((_MTOK, _NTOK)), whole((_NTOK, _MTOK)),
            whole((C, C)), whole((1, C)), whole((C, C)), whole((1, C)),
            whole((C, C)), whole((1, C)), whole((C, C)), whole((1, C)),
            pl.BlockSpec(memory_space=pltpu.SMEM),
            pl.BlockSpec(memory_space=pltpu.SMEM),
            pl.BlockSpec(memory_space=pltpu.SMEM),
        ],
        out_specs=[
            pl.BlockSpec((1, _NTOK, C), lambda b: (b, 0, 0)),
            pl.BlockSpec((1, _MTOK, C), lambda b: (b, 0, 0)),
            pl.BlockSpec((1, 1, 16), lambda b: (b, 0, 0)),
        ],
        out_shape=[
            jax.ShapeDtypeStruct((B, _NTOK, C), f32),
            jax.ShapeDtypeStruct((B, _MTOK, C), f32),
            jax.ShapeDtypeStruct((B, 1, 16), f32),
        ],
        compiler_params=_vm(),
        name="final_attn",
    )(vis_r, ir_r, lan_t, pimg, ptxt, mvd, mtd,
      W1, b1, W2, b2, W4, W5,
      ml_w, ml_b.reshape(1, C), ml2_w, ml2_b.reshape(1, C),
      vout_w, vout_b.reshape(1, C), tout_w, tout_b.reshape(1, C),
      sigma, sigma2, scalars)

    new_vis = vis_tok.transpose(0, 2, 1).reshape(B, C, 10, 10)

    p = parts[:, 0, :]
    nA = B * _MTOK * _NTOK
    nB = B * _NTOK * _MTOK
    l1 = jnp.sum(p[:, 0]) / nA + jnp.sum(p[:, 1]) / nB
    var = lambda sx, sx2, n: (jnp.sum(sx2) - jnp.sum(sx) ** 2 / n) / (n - 1)
    l2 = -jnp.abs(var(p[:, 2], p[:, 3], nA) - var(p[:, 4], p[:, 5], nA)) \
         - jnp.abs(var(p[:, 6], p[:, 7], nB) - var(p[:, 8], p[:, 9], nB))
    loss = (l1 + 100.0 * l2) * 1000.0
    return new_vis, new_lan, loss
